# baseline (speedup 1.0000x reference)
import functools
import jax
import jax.numpy as jnp
from jax import lax
from jax.experimental import pallas as pl
from jax.experimental.pallas import tpu as pltpu
from jax.experimental.pallas import tpu_sc as plsc

_N = 4096
_LANES = 16
_NW = 32
_ROWS_PER_W = _N // _NW
_CHUNK = 1
_NBUF = 8
_NCHUNK = _ROWS_PER_W // _CHUNK
_GROUPS = _NCHUNK // _NBUF
_VECS = _N // _LANES
_UNROLL = 4

_GATHER_DNUMS = lax.GatherDimensionNumbers(
    offset_dims=(), collapsed_slice_dims=(0,), start_index_map=(0,))


def _sc_body(gamma_hbm, s_hbm, t_hbm, out_hbm,
             gamma_v, s_bufs, t_bufs, o_bufs, in_sems, out_sems):
    wid = lax.axis_index("s") * 2 + lax.axis_index("c")
    row0 = wid * _ROWS_PER_W

    pltpu.sync_copy(gamma_hbm, gamma_v)
    alpha = 1.0 / (1.0 + jnp.exp(-gamma_v[...]))
    beta = 1.0 - alpha
    iota = lax.iota(jnp.int32, _LANES)

    def in_copy(chunk, b):
        base = row0 + jnp.minimum(chunk, _NCHUNK - 1) * _CHUNK
        cs = pltpu.async_copy(s_hbm.at[pl.ds(base, _CHUNK)], s_bufs.at[b],
                              in_sems.at[b])
        ct = pltpu.async_copy(t_hbm.at[pl.ds(base, _CHUNK)], t_bufs.at[b],
                              in_sems.at[b])
        return cs, ct

    def in_wait(b):
        pltpu.make_async_copy(s_hbm.at[pl.ds(row0, _CHUNK)], s_bufs.at[b],
                              in_sems.at[b]).wait()
        pltpu.make_async_copy(t_hbm.at[pl.ds(row0, _CHUNK)], t_bufs.at[b],
                              in_sems.at[b]).wait()

    def out_wait(b):
        pltpu.make_async_copy(o_bufs.at[b], out_hbm.at[pl.ds(row0, _CHUNK)],
                              out_sems.at[b]).wait()

    for b in range(_NBUF):
        in_copy(jnp.int32(b), b)

    def group_body(g, carry):
        for b in range(_NBUF):
            c = g * _NBUF + b
            base = row0 + c * _CHUNK
            in_wait(b)

            @pl.when(g >= 1)
            def _():
                out_wait(b)

            for r in range(_CHUNK):
                def blend_body(jj, acc):
                    for u in range(_UNROLL):
                        sl = pl.ds((jj * _UNROLL + u) * _LANES, _LANES)
                        av = alpha * s_bufs[b, r, sl] + beta * t_bufs[b, r, sl]
                        o_bufs[b, r, sl] = av
                        acc = acc + av
                    return acc

                acc = lax.fori_loop(0, _VECS // _UNROLL, blend_body,
                                    jnp.zeros((_LANES,), jnp.float32))
                for k in (1, 2, 4, 8):
                    perm = jnp.bitwise_xor(iota, k)
                    acc = acc + lax.gather(
                        acc, perm[:, None], _GATHER_DNUMS, slice_sizes=(1,),
                        mode=lax.GatherScatterMode.PROMISE_IN_BOUNDS)
                total = jnp.where(acc == 0.0, 1.0, acc)
                scale = 1.0 / total

                def scale_body(jj, c2):
                    for u in range(_UNROLL):
                        sl = pl.ds((jj * _UNROLL + u) * _LANES, _LANES)
                        o_bufs[b, r, sl] = o_bufs[b, r, sl] * scale
                    return c2

                lax.fori_loop(0, _VECS // _UNROLL, scale_body, 0)

            pltpu.async_copy(o_bufs.at[b], out_hbm.at[pl.ds(base, _CHUNK)],
                             out_sems.at[b])
            in_copy(c + _NBUF, b)
        return carry

    lax.fori_loop(0, _GROUPS, group_body, 0)

    for b in range(_NBUF):
        in_wait(b)
        out_wait(b)


def kernel(A_s, A_t, gamma):
    n, m = A_s.shape
    gamma_arr = jnp.broadcast_to(jnp.reshape(gamma, (1,)), (_LANES,)).astype(
        jnp.float32)
    mesh = plsc.VectorSubcoreMesh(core_axis_name="c", subcore_axis_name="s")
    run = functools.partial(
        pl.kernel,
        out_type=jax.ShapeDtypeStruct((n, m), jnp.float32),
        mesh=mesh,
        scratch_types=[
            pltpu.VMEM((_LANES,), jnp.float32),
            pltpu.VMEM((_NBUF, _CHUNK, _N), jnp.float32),
            pltpu.VMEM((_NBUF, _CHUNK, _N), jnp.float32),
            pltpu.VMEM((_NBUF, _CHUNK, _N), jnp.float32),
            pltpu.SemaphoreType.DMA((_NBUF,)),
            pltpu.SemaphoreType.DMA((_NBUF,)),
        ],
    )(_sc_body)
    return run(gamma_arr, A_s, A_t)


# speedup vs baseline: 1.1473x; 1.1473x over previous
import functools
import jax
import jax.numpy as jnp
from jax import lax
from jax.experimental import pallas as pl
from jax.experimental.pallas import tpu as pltpu
from jax.experimental.pallas import tpu_sc as plsc

_N = 4096
_LANES = 16
_NW = 32
_ROWS_PER_W = _N // _NW
_CHUNK = 1
_NBUF = 8
_NCHUNK = _ROWS_PER_W // _CHUNK
_GROUPS = _NCHUNK // _NBUF
_VECS = _N // _LANES
_UNROLL = 8

_GATHER_DNUMS = lax.GatherDimensionNumbers(
    offset_dims=(), collapsed_slice_dims=(0,), start_index_map=(0,))


def _sc_body(gamma_hbm, s_hbm, t_hbm, out_hbm,
             gamma_v, s_bufs, t_bufs, o_bufs, in_sems, out_sems, gamma_sem):
    wid = lax.axis_index("s") * 2 + lax.axis_index("c")
    row0 = wid * _ROWS_PER_W

    gamma_cp = pltpu.async_copy(gamma_hbm, gamma_v, gamma_sem)

    def in_copy(chunk, b):
        base = row0 + jnp.minimum(chunk, _NCHUNK - 1) * _CHUNK
        cs = pltpu.async_copy(s_hbm.at[pl.ds(base, _CHUNK)], s_bufs.at[b],
                              in_sems.at[b])
        ct = pltpu.async_copy(t_hbm.at[pl.ds(base, _CHUNK)], t_bufs.at[b],
                              in_sems.at[b])
        return cs, ct

    def in_wait(b):
        pltpu.make_async_copy(s_hbm.at[pl.ds(row0, _CHUNK)], s_bufs.at[b],
                              in_sems.at[b]).wait()
        pltpu.make_async_copy(t_hbm.at[pl.ds(row0, _CHUNK)], t_bufs.at[b],
                              in_sems.at[b]).wait()

    def out_wait(b):
        pltpu.make_async_copy(o_bufs.at[b], out_hbm.at[pl.ds(row0, _CHUNK)],
                              out_sems.at[b]).wait()

    for b in range(_NBUF):
        in_copy(jnp.int32(b), b)

    gamma_cp.wait()
    alpha = 1.0 / (1.0 + jnp.exp(-gamma_v[...]))
    beta = 1.0 - alpha
    iota = lax.iota(jnp.int32, _LANES)

    def group_body(g, carry):
        for b in range(_NBUF):
            c = g * _NBUF + b
            base = row0 + c * _CHUNK
            in_wait(b)

            @pl.when(g >= 1)
            def _():
                out_wait(b)

            for r in range(_CHUNK):
                def blend_body(jj, acc):
                    for u in range(_UNROLL):
                        sl = pl.ds((jj * _UNROLL + u) * _LANES, _LANES)
                        av = alpha * s_bufs[b, r, sl] + beta * t_bufs[b, r, sl]
                        o_bufs[b, r, sl] = av
                        acc = acc + av
                    return acc

                acc = lax.fori_loop(0, _VECS // _UNROLL, blend_body,
                                    jnp.zeros((_LANES,), jnp.float32))
                for k in (1, 2, 4, 8):
                    perm = jnp.bitwise_xor(iota, k)
                    acc = acc + lax.gather(
                        acc, perm[:, None], _GATHER_DNUMS, slice_sizes=(1,),
                        mode=lax.GatherScatterMode.PROMISE_IN_BOUNDS)
                total = jnp.where(acc == 0.0, 1.0, acc)
                scale = 1.0 / total

                def scale_body(jj, c2):
                    for u in range(_UNROLL):
                        sl = pl.ds((jj * _UNROLL + u) * _LANES, _LANES)
                        o_bufs[b, r, sl] = o_bufs[b, r, sl] * scale
                    return c2

                lax.fori_loop(0, _VECS // _UNROLL, scale_body, 0)

            pltpu.async_copy(o_bufs.at[b], out_hbm.at[pl.ds(base, _CHUNK)],
                             out_sems.at[b])
            in_copy(c + _NBUF, b)
        return carry

    lax.fori_loop(0, _GROUPS, group_body, 0)

    for b in range(_NBUF):
        in_wait(b)
        out_wait(b)


def kernel(A_s, A_t, gamma):
    n, m = A_s.shape
    gamma_arr = jnp.broadcast_to(jnp.reshape(gamma, (1,)), (_LANES,)).astype(
        jnp.float32)
    mesh = plsc.VectorSubcoreMesh(core_axis_name="c", subcore_axis_name="s")
    run = functools.partial(
        pl.kernel,
        out_type=jax.ShapeDtypeStruct((n, m), jnp.float32),
        mesh=mesh,
        scratch_types=[
            pltpu.VMEM((_LANES,), jnp.float32),
            pltpu.VMEM((_NBUF, _CHUNK, _N), jnp.float32),
            pltpu.VMEM((_NBUF, _CHUNK, _N), jnp.float32),
            pltpu.VMEM((_NBUF, _CHUNK, _N), jnp.float32),
            pltpu.SemaphoreType.DMA((_NBUF,)),
            pltpu.SemaphoreType.DMA((_NBUF,)),
            pltpu.SemaphoreType.DMA,
        ],
    )(_sc_body)
    return run(gamma_arr, A_s, A_t)
